# baseline (speedup 1.0000x reference)
import functools

import jax
import jax.numpy as jnp
from jax import lax
from jax.experimental import pallas as pl
from jax.experimental.pallas import tpu as pltpu
from jax.experimental.pallas import tpu_sc as plsc

_EMBED = 32
_PAD = 128
_B = 16384 * 26
_NC = 2
_NS = 16
_NW = _NC * _NS
_BPW = _B // _NW
_CHUNK = 416
_NCHUNK = _BPW // _CHUNK

_mesh = plsc.VectorSubcoreMesh(core_axis_name="c", subcore_axis_name="s")


@functools.partial(
    pl.kernel,
    mesh=_mesh,
    out_type=jax.ShapeDtypeStruct((_B, _EMBED), jnp.float32),
    scratch_types=[
        pltpu.VMEM((_NCHUNK, _CHUNK), jnp.int32),
        pltpu.VMEM((2, _CHUNK, _PAD), jnp.float32),
        pltpu.SemaphoreType.DMA,
        pltpu.SemaphoreType.DMA,
    ],
    compiler_params=pltpu.CompilerParams(use_tc_tiling_on_sc=False),
)
def _gather(idx_hbm, table_hbm, out_hbm, idx_v, rows_v, sem0, sem1):
    wid = lax.axis_index("s") * _NC + lax.axis_index("c")
    base = wid * _BPW
    sems = (sem0, sem1)
    pltpu.sync_copy(idx_hbm.at[pl.ds(wid * _NCHUNK, _NCHUNK)], idx_v)
    pltpu.async_copy(table_hbm.at[idx_v.at[0]], rows_v.at[0], sems[0])
    pltpu.async_copy(table_hbm.at[idx_v.at[1]], rows_v.at[1], sems[1])

    def body(i, carry):
        for b in range(2):
            j = 2 * i + b
            pltpu.make_async_copy(
                table_hbm.at[idx_v.at[j]], rows_v.at[b], sems[b]).wait()
            pltpu.sync_copy(
                rows_v.at[b, :, pl.ds(0, _EMBED)],
                out_hbm.at[pl.ds(base + j * _CHUNK, _CHUNK)])

            @pl.when(j + 2 < _NCHUNK)
            def _():
                pltpu.async_copy(
                    table_hbm.at[idx_v.at[j + 2]], rows_v.at[b], sems[b])
        return carry

    lax.fori_loop(0, _NCHUNK // 2, body, 0)


def kernel(x, table):
    idx = x.reshape(_B // _CHUNK, _CHUNK).astype(jnp.int32)
    tp = jnp.pad(table, ((0, 0), (0, _PAD - _EMBED)))
    out = _gather(idx, tp)
    return out.reshape(x.shape + (_EMBED,))
